# baseline (speedup 1.0000x reference)
import jax
import jax.numpy as jnp
from jax.experimental import pallas as pl

_N = 256
_BB_MIN = -2.0
_BB_MAX = 2.0
_VOXEL_SIZE = (_BB_MAX - _BB_MIN) / (_N - 1)
_INV2VS = 1.0 / (2.0 * _VOXEL_SIZE)

_BX = 16
_NUM_BLOCKS = _N // _BX


def _normals_body(c_ref, ph_ref, nh_ref, o_ref):
    inv = jnp.float32(_INV2VS)

    for r in range(_BX):
        row = c_ref[r]

        row_m = ph_ref[0] if r == 0 else c_ref[r - 1]
        row_p = nh_ref[0] if r == _BX - 1 else c_ref[r + 1]
        o_ref[0, r] = (row_p - row_m) * inv

        y0 = row[1:2, :] - 1.5 * row[0:1, :] + 0.5 * row[2:3, :]
        y_int = row[2:, :] - row[:-2, :]
        yn = 1.5 * row[-1:, :] - row[-2:-1, :] - 0.5 * row[-3:-2, :]
        o_ref[1, r] = jnp.concatenate([y0, y_int, yn], axis=0) * inv

        z0 = row[:, 1:2] - 1.5 * row[:, 0:1] + 0.5 * row[:, 2:3]
        z_int = row[:, 2:] - row[:, :-2]
        zn = 1.5 * row[:, -1:] - row[:, -2:-1] - 0.5 * row[:, -3:-2]
        o_ref[2, r] = jnp.concatenate([z0, z_int, zn], axis=1) * inv

    i = pl.program_id(0)

    @pl.when(i == 0)
    def _fix_first():
        o_ref[0, 0] = (c_ref[1] - 1.5 * c_ref[0] + 0.5 * c_ref[2]) * inv

    @pl.when(i == _NUM_BLOCKS - 1)
    def _fix_last():
        o_ref[0, _BX - 1] = (
            1.5 * c_ref[_BX - 1]
            - c_ref[_BX - 2]
            - 0.5 * c_ref[_BX - 3]
        ) * inv


def kernel(grid):
    return pl.pallas_call(
        _normals_body,
        grid=(_NUM_BLOCKS,),
        in_specs=[
            pl.BlockSpec((_BX, _N, _N), lambda i: (i, 0, 0)),
            pl.BlockSpec(
                (1, _N, _N), lambda i: (jnp.maximum(i * _BX - 1, 0), 0, 0)
            ),
            pl.BlockSpec(
                (1, _N, _N),
                lambda i: (jnp.minimum(i * _BX + _BX, _N - 1), 0, 0),
            ),
        ],
        out_specs=pl.BlockSpec((3, _BX, _N, _N), lambda i: (0, i, 0, 0)),
        out_shape=jax.ShapeDtypeStruct((3, _N, _N, _N), jnp.float32),
    )(grid, grid, grid)
